# Initial kernel scaffold; baseline (speedup 1.0000x reference)
import jax
import jax.numpy as jnp
from jax.experimental import pallas as pl


def kernel(x, noise_labels, map0_w, map0_b, map1_w, map1_b, out_w, out_b, block0_lin_w, block0_lin_b, block0_gn_w, block0_gn_b, block0_qkv_w, block0_qkv_b, block0_proj_w, block0_proj_b, block1_lin_w, block1_lin_b, block1_gn_w, block1_gn_b, block1_qkv_w, block1_qkv_b, block1_proj_w, block1_proj_b):
    raise NotImplementedError("write your pallas kernel here")


# speedup vs baseline: 1.9745x; 1.9745x over previous
import functools
import math

import jax
import jax.numpy as jnp
from jax.experimental import pallas as pl
from jax.experimental.pallas import tpu as pltpu

_EPS = 1e-6
_MAX_POSITIONS = 10000.0


def _fused_kernel(emb0_ref, x_ref,
                  m0_ref, mb0_ref, m1_ref, mb1_ref, A_ref,
                  wx0_ref, we0_ref, lb0_ref, gw0_ref, gb0_ref,
                  wq0_ref, bq0_ref, wk0_ref, bk0_ref, wv0_ref, bv0_ref,
                  wp0_ref, bp0_ref,
                  wx1_ref, we1_ref, lb1_ref, gw1_ref, gb1_ref,
                  wq1_ref, bq1_ref, wk1_ref, bk1_ref, wv1_ref, bv1_ref,
                  wp1_ref, bp1_ref,
                  wo_ref, bo_ref, o_ref, *, bt, t, inv_gd, scale):
    f32 = jnp.float32
    bf16 = jnp.bfloat16

    def mm(a, w_ref):
        return jax.lax.dot_general(a.astype(bf16), w_ref[...],
                                   (((1,), (0,)), ((), ())),
                                   preferred_element_type=f32)

    e = mm(emb0_ref[0], m0_ref) + mb0_ref[...]
    e = e * jax.nn.sigmoid(e)
    e = mm(e, m1_ref) + mb1_ref[...]
    emb = e * jax.nn.sigmoid(e)

    A = A_ref[...]

    def linear_block(xf, we_ref, lb_ref, wx_ref, gw_ref, gb_ref,
                     wq_ref, bq_ref, wk_ref, bk_ref, wv_ref, bv_ref,
                     wp_ref, bp_ref):
        embc = mm(emb, we_ref) + lb_ref[...]
        h = mm(xf, wx_ref)
        c_dim = h.shape[1]
        h = h.reshape(bt, t, c_dim) + embc[:, None, :]
        h = jnp.maximum(h, 0.0).reshape(bt * t, c_dim)

        gw = gw_ref[...]
        gb = gb_ref[...]
        parts = []
        for b in range(bt):
            hb = h[b * t:(b + 1) * t]
            s1 = jnp.sum(hb, axis=1, keepdims=True)
            mu = jax.lax.dot_general(A, s1, (((1,), (0,)), ((), ())),
                                     preferred_element_type=f32) * inv_gd
            cen = hb - mu
            s2 = jnp.sum(cen * cen, axis=1, keepdims=True)
            var = jax.lax.dot_general(A, s2, (((1,), (0,)), ((), ())),
                                      preferred_element_type=f32) * inv_gd
            parts.append(cen * jax.lax.rsqrt(var + _EPS) * gw + gb)
        hn = jnp.concatenate(parts, axis=0)

        q = mm(hn, wq_ref) + bq_ref[...]
        k = mm(hn, wk_ref) + bk_ref[...]
        v = mm(hn, wv_ref) + bv_ref[...]

        outs = []
        for b in range(bt):
            qb = q[b * t:(b + 1) * t].astype(bf16)
            kb = k[b * t:(b + 1) * t].astype(bf16)
            logits = jax.lax.dot_general(qb, kb, (((1,), (1,)), ((), ())),
                                         preferred_element_type=f32) * scale
            m = jnp.max(logits, axis=-1, keepdims=True)
            p = jnp.exp(logits - m)
            w = (p / jnp.sum(p, axis=-1, keepdims=True)).astype(bf16)
            outs.append(jax.lax.dot_general(
                w, v[b * t:(b + 1) * t].astype(bf16),
                (((1,), (0,)), ((), ())), preferred_element_type=f32))
        a = jnp.concatenate(outs, axis=0)
        return mm(a, wp_ref) + bp_ref[...] + hn

    xin = x_ref[...]
    d_in = xin.shape[2]
    x1 = linear_block(xin.reshape(bt * t, d_in),
                      we0_ref, lb0_ref, wx0_ref, gw0_ref, gb0_ref,
                      wq0_ref, bq0_ref, wk0_ref, bk0_ref, wv0_ref, bv0_ref,
                      wp0_ref, bp0_ref)
    x2 = linear_block(x1,
                      we1_ref, lb1_ref, wx1_ref, gw1_ref, gb1_ref,
                      wq1_ref, bq1_ref, wk1_ref, bk1_ref, wv1_ref, bv1_ref,
                      wp1_ref, bp1_ref)
    out = mm(x2, wo_ref) + bo_ref[...]
    o_ref[...] = out.reshape(bt, t, out.shape[1])


def kernel(x, noise_labels, map0_w, map0_b, map1_w, map1_b, out_w, out_b,
           block0_lin_w, block0_lin_b, block0_gn_w, block0_gn_b,
           block0_qkv_w, block0_qkv_b, block0_proj_w, block0_proj_b,
           block1_lin_w, block1_lin_b, block1_gn_w, block1_gn_b,
           block1_qkv_w, block1_qkv_b, block1_proj_w, block1_proj_b):
    B, T, D = x.shape
    E = map0_w.shape[0]
    NC = map0_w.shape[1]
    C = block0_lin_w.shape[0]
    Dout = out_w.shape[0]
    num_groups = min(8, T // 4)
    group_rows = T // num_groups
    inv_gd = 1.0 / (group_rows * C)
    scale = 1.0 / math.sqrt(C)
    bf16 = jnp.bfloat16

    F = NC // 2
    freqs = (1.0 / _MAX_POSITIONS) ** (
        jnp.arange(F, dtype=jnp.float32) / (F - 1))
    phase = noise_labels[:, None].astype(jnp.float32) * freqs[None, :]
    emb0 = jnp.concatenate([jnp.sin(phase), jnp.cos(phase)], axis=1)

    BT = next(bt for bt in (4, 3, 2, 1) if B % bt == 0)
    steps = B // BT
    emb0r = emb0.reshape(steps, BT, NC)

    gidx = jnp.arange(T) // group_rows
    A = (gidx[:, None] == gidx[None, :]).astype(jnp.float32)

    def prep_block(lw, lb, gw, gb, qkvw, qkvb, pw, pb, cin):
        return [lw[:, :cin].T.astype(bf16), lw[:, cin:].T.astype(bf16),
                lb.reshape(1, C),
                gw.reshape(T, 1), gb.reshape(T, 1),
                qkvw[:C].T.astype(bf16), qkvb[:C].reshape(1, C),
                qkvw[C:2 * C].T.astype(bf16), qkvb[C:2 * C].reshape(1, C),
                qkvw[2 * C:].T.astype(bf16), qkvb[2 * C:].reshape(1, C),
                pw.T.astype(bf16), pb.reshape(1, C)]

    args = ([emb0r, x,
             map0_w.T.astype(bf16), map0_b.reshape(1, E),
             map1_w.T.astype(bf16), map1_b.reshape(1, E), A]
            + prep_block(block0_lin_w, block0_lin_b, block0_gn_w, block0_gn_b,
                         block0_qkv_w, block0_qkv_b,
                         block0_proj_w, block0_proj_b, D)
            + prep_block(block1_lin_w, block1_lin_b, block1_gn_w, block1_gn_b,
                         block1_qkv_w, block1_qkv_b,
                         block1_proj_w, block1_proj_b, C)
            + [out_w.T.astype(bf16), out_b.reshape(1, Dout)])

    def const_spec(a):
        nd = a.ndim
        return pl.BlockSpec(a.shape, lambda i, _nd=nd: (0,) * _nd)

    in_specs = ([pl.BlockSpec((1, BT, NC), lambda i: (i, 0, 0)),
                 pl.BlockSpec((BT, T, D), lambda i: (i, 0, 0))]
                + [const_spec(a) for a in args[2:]])

    fn = functools.partial(_fused_kernel, bt=BT, t=T,
                           inv_gd=inv_gd, scale=scale)
    out = pl.pallas_call(
        fn,
        grid=(steps,),
        in_specs=in_specs,
        out_specs=pl.BlockSpec((BT, T, Dout), lambda i: (i, 0, 0)),
        out_shape=jax.ShapeDtypeStruct((B, T, Dout), jnp.float32),
        compiler_params=pltpu.CompilerParams(
            dimension_semantics=("parallel",)),
    )(*args)
    return out


# speedup vs baseline: 2.4986x; 1.2654x over previous
import functools
import math

import jax
import jax.numpy as jnp
from jax.experimental import pallas as pl
from jax.experimental.pallas import tpu as pltpu

_EPS = 1e-6
_MAX_POSITIONS = 10000.0


def _fused_kernel(emb0_ref, x_ref,
                  m0_ref, mb0_ref, m1_ref, mb1_ref,
                  Ag_ref, Mask_ref,
                  lin0_ref, lb0_ref, gw0_ref, gb0_ref,
                  qkv0_ref, qb0_ref, prj0_ref, pb0_ref,
                  lin1_ref, lb1_ref, gw1_ref, gb1_ref,
                  qkv1_ref, qb1_ref, prj1_ref, pb1_ref,
                  wo_ref, bo_ref, o_ref,
                  s_map0, s_map1, s_lin0, s_qkv0, s_prj0,
                  s_lin1, s_qkv1, s_prj1, s_out, s_gn,
                  *, bt, t, inv_gd, scale):
    f32 = jnp.float32
    bf16 = jnp.bfloat16
    C = prj0_ref.shape[0]

    @pl.when(pl.program_id(0) == 0)
    def _prep():
        s_map0[...] = m0_ref[...].T.astype(bf16)
        s_map1[...] = m1_ref[...].T.astype(bf16)
        s_lin0[...] = lin0_ref[...].T.astype(bf16)
        s_qkv0[...] = qkv0_ref[...].T.astype(bf16)
        s_prj0[...] = prj0_ref[...].T.astype(bf16)
        s_lin1[...] = lin1_ref[...].T.astype(bf16)
        s_qkv1[...] = qkv1_ref[...].T.astype(bf16)
        s_prj1[...] = prj1_ref[...].T.astype(bf16)
        s_out[...] = wo_ref[...].T.astype(bf16)
        gcols = jnp.concatenate([gw0_ref[...].T, gb0_ref[...].T,
                                 gw1_ref[...].T, gb1_ref[...].T], axis=1)
        s_gn[...] = jnp.concatenate([gcols] * bt, axis=0)

    def mm(a, w):
        return jax.lax.dot_general(a.astype(bf16), w,
                                   (((1,), (0,)), ((), ())),
                                   preferred_element_type=f32)

    e = mm(emb0_ref[0], s_map0[...]) + mb0_ref[...]
    e = e * jax.nn.sigmoid(e)
    e = mm(e, s_map1[...]) + mb1_ref[...]
    emb = e * jax.nn.sigmoid(e)

    Ag = Ag_ref[...]
    Mask = Mask_ref[...]

    def linear_block(xf, s_lin, lb_ref, s_qkv, qb_ref, s_prj, pb_ref,
                     gw, gb, cin):
        wfull = s_lin[...]
        embc = mm(emb, wfull[cin:]) + lb_ref[...]
        h = mm(xf, wfull[:cin])
        h = h.reshape(bt, t, C) + embc[:, None, :]
        h = jnp.maximum(h, 0.0).reshape(bt * t, C)

        s1 = jnp.sum(h, axis=1, keepdims=True)
        s2 = jnp.sum(h * h, axis=1, keepdims=True)
        g = jax.lax.dot_general(Ag, jnp.concatenate([s1, s2], axis=1),
                                (((1,), (0,)), ((), ())),
                                preferred_element_type=f32) * inv_gd
        mu = g[:, 0:1]
        var = g[:, 1:2] - mu * mu
        hn = (h - mu) * jax.lax.rsqrt(var + _EPS) * gw + gb

        z = mm(hn, s_qkv[...]) + qb_ref[...]
        q = z[:, :C].astype(bf16)
        k = z[:, C:2 * C].astype(bf16)
        v = z[:, 2 * C:].astype(bf16)
        logits = jax.lax.dot_general(q, k, (((1,), (1,)), ((), ())),
                                     preferred_element_type=f32)
        logits = logits * scale + Mask
        m = jnp.max(logits, axis=-1, keepdims=True)
        p = jnp.exp(logits - m)
        w = (p / jnp.sum(p, axis=-1, keepdims=True)).astype(bf16)
        a = jax.lax.dot_general(w, v, (((1,), (0,)), ((), ())),
                                preferred_element_type=f32)
        return mm(a, s_prj[...]) + pb_ref[...] + hn

    xin = x_ref[...]
    d_in = xin.shape[2]
    x1 = linear_block(xin.reshape(bt * t, d_in), s_lin0, lb0_ref,
                      s_qkv0, qb0_ref, s_prj0, pb0_ref,
                      s_gn[:, 0:1], s_gn[:, 1:2], d_in)
    x2 = linear_block(x1, s_lin1, lb1_ref,
                      s_qkv1, qb1_ref, s_prj1, pb1_ref,
                      s_gn[:, 2:3], s_gn[:, 3:4], C)
    out = mm(x2, s_out[...]) + bo_ref[...]
    o_ref[...] = out.reshape(bt, t, out.shape[1])


def kernel(x, noise_labels, map0_w, map0_b, map1_w, map1_b, out_w, out_b,
           block0_lin_w, block0_lin_b, block0_gn_w, block0_gn_b,
           block0_qkv_w, block0_qkv_b, block0_proj_w, block0_proj_b,
           block1_lin_w, block1_lin_b, block1_gn_w, block1_gn_b,
           block1_qkv_w, block1_qkv_b, block1_proj_w, block1_proj_b):
    B, T, D = x.shape
    E = map0_w.shape[0]
    NC = map0_w.shape[1]
    C = block0_lin_w.shape[0]
    Dout = out_w.shape[0]
    num_groups = min(8, T // 4)
    group_rows = T // num_groups
    inv_gd = 1.0 / (group_rows * C)
    scale = 1.0 / math.sqrt(C)

    F = NC // 2
    freqs = (1.0 / _MAX_POSITIONS) ** (
        jnp.arange(F, dtype=jnp.float32) / (F - 1))
    phase = noise_labels[:, None].astype(jnp.float32) * freqs[None, :]
    emb0 = jnp.concatenate([jnp.sin(phase), jnp.cos(phase)], axis=1)

    BT = next(bt for bt in (4, 3, 2, 1) if B % bt == 0)
    steps = B // BT
    emb0r = emb0.reshape(steps, BT, NC)
    M = BT * T

    r = jnp.arange(M)
    Ag = (r[:, None] // group_rows == r[None, :] // group_rows
          ).astype(jnp.float32)
    Mask = jnp.where(r[:, None] // T == r[None, :] // T, 0.0, -1e30
                     ).astype(jnp.float32)

    args = [emb0r, x,
            map0_w, map0_b.reshape(1, E), map1_w, map1_b.reshape(1, E),
            Ag, Mask,
            block0_lin_w, block0_lin_b.reshape(1, C),
            block0_gn_w.reshape(1, T), block0_gn_b.reshape(1, T),
            block0_qkv_w, block0_qkv_b.reshape(1, 3 * C),
            block0_proj_w, block0_proj_b.reshape(1, C),
            block1_lin_w, block1_lin_b.reshape(1, C),
            block1_gn_w.reshape(1, T), block1_gn_b.reshape(1, T),
            block1_qkv_w, block1_qkv_b.reshape(1, 3 * C),
            block1_proj_w, block1_proj_b.reshape(1, C),
            out_w, out_b.reshape(1, Dout)]

    def const_spec(a):
        nd = a.ndim
        return pl.BlockSpec(a.shape, lambda i, _nd=nd: (0,) * _nd)

    in_specs = ([pl.BlockSpec((1, BT, NC), lambda i: (i, 0, 0)),
                 pl.BlockSpec((BT, T, D), lambda i: (i, 0, 0))]
                + [const_spec(a) for a in args[2:]])

    bf16 = jnp.bfloat16
    scratch_shapes = [
        pltpu.VMEM((NC, E), bf16),
        pltpu.VMEM((E, E), bf16),
        pltpu.VMEM((D + E, C), bf16),
        pltpu.VMEM((C, 3 * C), bf16),
        pltpu.VMEM((C, C), bf16),
        pltpu.VMEM((C + E, C), bf16),
        pltpu.VMEM((C, 3 * C), bf16),
        pltpu.VMEM((C, C), bf16),
        pltpu.VMEM((C, Dout), bf16),
        pltpu.VMEM((M, 4), jnp.float32),
    ]

    fn = functools.partial(_fused_kernel, bt=BT, t=T,
                           inv_gd=inv_gd, scale=scale)
    out = pl.pallas_call(
        fn,
        grid=(steps,),
        in_specs=in_specs,
        out_specs=pl.BlockSpec((BT, T, Dout), lambda i: (i, 0, 0)),
        out_shape=jax.ShapeDtypeStruct((B, T, Dout), jnp.float32),
        scratch_shapes=scratch_shapes,
        compiler_params=pltpu.CompilerParams(
            dimension_semantics=("arbitrary",)),
    )(*args)
    return out
